# Initial kernel scaffold; baseline (speedup 1.0000x reference)
import jax
import jax.numpy as jnp
from jax.experimental import pallas as pl


def kernel(pos, edge_index, pair_indices, source_elements, target_elements, inverse_distances_sq, edge_unit_vectors, A, B, raw_charges):
    raise NotImplementedError("write your pallas kernel here")


# speedup vs baseline: 1.4580x; 1.4580x over previous
import jax
import jax.numpy as jnp
from jax import lax
from jax.experimental import pallas as pl
from jax.experimental.pallas import tpu as pltpu
from jax.experimental.pallas import tpu_sc as plsc

N_NODES = 100000
N_EDGES = 3200000

SUB = 80
NSUB = 10
CHUNK = SUB * NSUB
N_CHUNKS = N_EDGES // CHUNK
CHUNKS_PER_W = N_CHUNKS // 32

ROWS_A = 6240
ROWS_LAST = N_NODES - 15 * ROWS_A


def _sc_scatter_kernel(dst1, pair, se, te, inv2, euv, a16, b16, cp16, out,
                       a_v, b_v, cp_v, dst1_v, dst_v,
                       pair_v, se_v, te_v, inv2_v, euv_v, s_v,
                       updx_v, updy_v, updz_v, zb_v,
                       accx, accy, accz, sem_in, sem_sc):
    f32, i32 = jnp.float32, jnp.int32
    cid = lax.axis_index("c")
    sid = lax.axis_index("s")
    wid = cid * 16 + sid
    row0 = sid * ROWS_A

    pltpu.sync_copy(a16, a_v)
    pltpu.sync_copy(b16, b_v)
    pltpu.sync_copy(cp16, cp_v)

    z16 = jnp.zeros((16,), f32)

    def zbody(k, carry):
        zb_v[pl.ds(k * 16, 16)] = z16
        return carry

    lax.fori_loop(0, ROWS_LAST // 16, zbody, 0)

    @pl.when(sid < 15)
    def _():
        for acc in (accx, accy, accz):
            pltpu.sync_copy(zb_v.at[pl.ds(0, ROWS_A)],
                            acc.at[pl.ds(row0, ROWS_A)])

    @pl.when(sid == 15)
    def _():
        for acc in (accx, accy, accz):
            pltpu.sync_copy(zb_v, acc.at[pl.ds(15 * ROWS_A, ROWS_LAST)])

    plsc.subcore_barrier()

    iota3 = lax.iota(i32, 16) * 3

    def body(i, carry):
        c = wid * CHUNKS_PER_W + i
        e0 = c * CHUNK
        cps = [
            pltpu.async_copy(dst1.at[pl.ds(e0, CHUNK)], dst1_v, sem_in),
            pltpu.async_copy(pair.at[pl.ds(e0, CHUNK)], pair_v, sem_in),
            pltpu.async_copy(se.at[pl.ds(e0, CHUNK)], se_v, sem_in),
            pltpu.async_copy(te.at[pl.ds(e0, CHUNK)], te_v, sem_in),
            pltpu.async_copy(inv2.at[pl.ds(e0, CHUNK)], inv2_v, sem_in),
            pltpu.async_copy(euv.at[pl.ds(e0 * 3, CHUNK * 3)], euv_v, sem_in),
        ]
        for cp in cps:
            cp.wait()

        for j in range(NSUB):
            for g in range(SUB // 16):
                sl = pl.ds(g * 16, 16)
                dst_v[j, sl] = dst1_v[pl.ds(j * SUB + g * 16, 16)]

        for g in range(CHUNK // 16):
            sl = pl.ds(g * 16, 16)
            pv = pair_v[sl]
            iv = inv2_v[sl]
            ci = se_v[sl] + se_v[sl] + te_v[sl]
            av = plsc.load_gather(a_v, [pv])
            bv = plsc.load_gather(b_v, [pv])
            cv = plsc.load_gather(cp_v, [ci])
            s_v[sl] = iv * (bv - cv - av * iv)

        descs = []
        for j in range(NSUB):
            for g in range(SUB // 16):
                sl = pl.ds(g * 16, 16)
                sle = pl.ds(j * SUB + g * 16, 16)
                sv = s_v[sle]
                eidx = (j * SUB + g * 16) * 3 + iota3
                exv = plsc.load_gather(euv_v, [eidx])
                eyv = plsc.load_gather(euv_v, [eidx + 1])
                ezv = plsc.load_gather(euv_v, [eidx + 2])
                updx_v[j, sl] = sv * exv
                updy_v[j, sl] = sv * eyv
                updz_v[j, sl] = sv * ezv
            idx = dst_v.at[j]
            descs.append(pltpu.async_copy(updx_v.at[j], accx.at[idx],
                                          sem_sc, add=True))
            descs.append(pltpu.async_copy(updy_v.at[j], accy.at[idx],
                                          sem_sc, add=True))
            descs.append(pltpu.async_copy(updz_v.at[j], accz.at[idx],
                                          sem_sc, add=True))
        for d in descs:
            d.wait()
        return carry

    lax.fori_loop(0, CHUNKS_PER_W, body, 0)
    plsc.subcore_barrier()

    for m, acc in enumerate((accx, accy, accz)):
        obase = m * (2 * N_NODES) + cid * N_NODES + row0

        @pl.when(sid < 15)
        def _(acc=acc, obase=obase):
            pltpu.sync_copy(acc.at[pl.ds(row0, ROWS_A)],
                            zb_v.at[pl.ds(0, ROWS_A)])
            pltpu.sync_copy(zb_v.at[pl.ds(0, ROWS_A)],
                            out.at[pl.ds(obase, ROWS_A)])

        @pl.when(sid == 15)
        def _(acc=acc, obase=obase):
            pltpu.sync_copy(acc.at[pl.ds(15 * ROWS_A, ROWS_LAST)], zb_v)
            pltpu.sync_copy(zb_v, out.at[pl.ds(obase, ROWS_LAST)])


def _add_body(a_ref, b_ref, o_ref):
    o_ref[...] = a_ref[...] + b_ref[...]


@jax.jit
def kernel(pos, edge_index, pair_indices, source_elements, target_elements,
           inverse_distances_sq, edge_unit_vectors, A, B, raw_charges):
    f32 = jnp.float32
    dst1 = edge_index[0]
    euv = edge_unit_vectors.reshape(-1)

    q = raw_charges[0]
    q2 = q * q
    cp16 = jnp.zeros((16,), f32).at[:4].set(jnp.stack([q2, -q2, -q2, q2]))
    a16 = jnp.zeros((16,), f32).at[:3].set(A)
    b16 = jnp.zeros((16,), f32).at[:3].set(B)

    mesh = plsc.VectorSubcoreMesh(core_axis_name="c", subcore_axis_name="s")
    partials = pl.kernel(
        _sc_scatter_kernel,
        out_type=jax.ShapeDtypeStruct((3 * 2 * N_NODES,), f32),
        mesh=mesh,
        compiler_params=pltpu.CompilerParams(needs_layout_passes=False),
        scratch_types=[
            pltpu.VMEM((16,), f32),
            pltpu.VMEM((16,), f32),
            pltpu.VMEM((16,), f32),
            pltpu.VMEM((CHUNK,), jnp.int32),
            pltpu.VMEM((NSUB, SUB), jnp.int32),
            pltpu.VMEM((CHUNK,), jnp.int32),
            pltpu.VMEM((CHUNK,), jnp.int32),
            pltpu.VMEM((CHUNK,), jnp.int32),
            pltpu.VMEM((CHUNK,), f32),
            pltpu.VMEM((CHUNK * 3,), f32),
            pltpu.VMEM((CHUNK,), f32),
            pltpu.VMEM((NSUB, SUB), f32),
            pltpu.VMEM((NSUB, SUB), f32),
            pltpu.VMEM((NSUB, SUB), f32),
            pltpu.VMEM((ROWS_LAST,), f32),
            pltpu.VMEM_SHARED((N_NODES,), f32),
            pltpu.VMEM_SHARED((N_NODES,), f32),
            pltpu.VMEM_SHARED((N_NODES,), f32),
            pltpu.SemaphoreType.DMA,
            pltpu.SemaphoreType.DMA,
        ],
    )(dst1, pair_indices, source_elements, target_elements,
      inverse_distances_sq, euv, a16, b16, cp16)

    p = partials.reshape(3, 2, N_NODES)
    summed = pl.pallas_call(
        _add_body,
        out_shape=jax.ShapeDtypeStruct((3, N_NODES), f32),
    )(p[:, 0, :], p[:, 1, :])
    return summed.T


# speedup vs baseline: 41.3226x; 28.3426x over previous
import jax
import jax.numpy as jnp
from jax import lax
from jax.experimental import pallas as pl
from jax.experimental.pallas import tpu as pltpu
from jax.experimental.pallas import tpu_sc as plsc

N_NODES = 100000
N_EDGES = 3200000

SUB = 80
NSUB = 10
CHUNK = SUB * NSUB
N_CHUNKS = N_EDGES // CHUNK
CHUNKS_PER_W = N_CHUNKS // 32

ROWS_A = 6240
ROWS_LAST = N_NODES - 15 * ROWS_A


def _sc_scatter_kernel(dst1, pair, se, te, inv2, ex, ey, ez, a16, b16, cp16,
                       out,
                       a_v, b_v, cp_v, dst1_v, dst_v,
                       pair_v, se_v, te_v, inv2_v, ex_v, ey_v, ez_v, s_v,
                       updx_v, updy_v, updz_v, zb_v,
                       accx, accy, accz, sem_in, sem_sc):
    f32, i32 = jnp.float32, jnp.int32
    cid = lax.axis_index("c")
    sid = lax.axis_index("s")
    wid = cid * 16 + sid
    row0 = sid * ROWS_A

    pltpu.sync_copy(a16, a_v)
    pltpu.sync_copy(b16, b_v)
    pltpu.sync_copy(cp16, cp_v)

    z16 = jnp.zeros((16,), f32)

    def zbody(k, carry):
        zb_v[pl.ds(k * 16, 16)] = z16
        return carry

    lax.fori_loop(0, ROWS_LAST // 16, zbody, 0)

    @pl.when(sid < 15)
    def _():
        for acc in (accx, accy, accz):
            pltpu.sync_copy(zb_v.at[pl.ds(0, ROWS_A)],
                            acc.at[pl.ds(row0, ROWS_A)])

    @pl.when(sid == 15)
    def _():
        for acc in (accx, accy, accz):
            pltpu.sync_copy(zb_v, acc.at[pl.ds(15 * ROWS_A, ROWS_LAST)])

    plsc.subcore_barrier()

    def body(i, carry):
        c = wid * CHUNKS_PER_W + i
        e0 = c * CHUNK
        cps = [
            pltpu.async_copy(dst1.at[pl.ds(e0, CHUNK)], dst1_v, sem_in),
            pltpu.async_copy(pair.at[pl.ds(e0, CHUNK)], pair_v, sem_in),
            pltpu.async_copy(se.at[pl.ds(e0, CHUNK)], se_v, sem_in),
            pltpu.async_copy(te.at[pl.ds(e0, CHUNK)], te_v, sem_in),
            pltpu.async_copy(inv2.at[pl.ds(e0, CHUNK)], inv2_v, sem_in),
            pltpu.async_copy(ex.at[pl.ds(e0, CHUNK)], ex_v, sem_in),
            pltpu.async_copy(ey.at[pl.ds(e0, CHUNK)], ey_v, sem_in),
            pltpu.async_copy(ez.at[pl.ds(e0, CHUNK)], ez_v, sem_in),
        ]
        for cp in cps:
            cp.wait()

        for j in range(NSUB):
            for g in range(SUB // 16):
                sl = pl.ds(g * 16, 16)
                dst_v[j, sl] = dst1_v[pl.ds(j * SUB + g * 16, 16)]

        for g in range(CHUNK // 16):
            sl = pl.ds(g * 16, 16)
            pv = pair_v[sl]
            iv = inv2_v[sl]
            ci = se_v[sl] + se_v[sl] + te_v[sl]
            av = plsc.load_gather(a_v, [pv])
            bv = plsc.load_gather(b_v, [pv])
            cv = plsc.load_gather(cp_v, [ci])
            s_v[sl] = iv * (bv - cv - av * iv)

        descs = []
        for j in range(NSUB):
            for g in range(SUB // 16):
                sl = pl.ds(g * 16, 16)
                sle = pl.ds(j * SUB + g * 16, 16)
                sv = s_v[sle]
                updx_v[j, sl] = sv * ex_v[sle]
                updy_v[j, sl] = sv * ey_v[sle]
                updz_v[j, sl] = sv * ez_v[sle]
            idx = dst_v.at[j]
            descs.append(pltpu.async_copy(updx_v.at[j], accx.at[idx],
                                          sem_sc, add=True))
            descs.append(pltpu.async_copy(updy_v.at[j], accy.at[idx],
                                          sem_sc, add=True))
            descs.append(pltpu.async_copy(updz_v.at[j], accz.at[idx],
                                          sem_sc, add=True))
        for d in descs:
            d.wait()
        return carry

    lax.fori_loop(0, CHUNKS_PER_W, body, 0)
    plsc.subcore_barrier()

    for m, acc in enumerate((accx, accy, accz)):
        obase = m * (2 * N_NODES) + cid * N_NODES + row0

        @pl.when(sid < 15)
        def _(acc=acc, obase=obase):
            pltpu.sync_copy(acc.at[pl.ds(row0, ROWS_A)],
                            zb_v.at[pl.ds(0, ROWS_A)])
            pltpu.sync_copy(zb_v.at[pl.ds(0, ROWS_A)],
                            out.at[pl.ds(obase, ROWS_A)])

        @pl.when(sid == 15)
        def _(acc=acc, obase=obase):
            pltpu.sync_copy(acc.at[pl.ds(15 * ROWS_A, ROWS_LAST)], zb_v)
            pltpu.sync_copy(zb_v, out.at[pl.ds(obase, ROWS_LAST)])


def _add_body(a_ref, b_ref, o_ref):
    o_ref[...] = a_ref[...] + b_ref[...]


@jax.jit
def kernel(pos, edge_index, pair_indices, source_elements, target_elements,
           inverse_distances_sq, edge_unit_vectors, A, B, raw_charges):
    f32 = jnp.float32
    dst1 = edge_index[0]
    ex = edge_unit_vectors[:, 0]
    ey = edge_unit_vectors[:, 1]
    ez = edge_unit_vectors[:, 2]

    q = raw_charges[0]
    q2 = q * q
    cp16 = jnp.zeros((16,), f32).at[:4].set(jnp.stack([q2, -q2, -q2, q2]))
    a16 = jnp.zeros((16,), f32).at[:3].set(A)
    b16 = jnp.zeros((16,), f32).at[:3].set(B)

    mesh = plsc.VectorSubcoreMesh(core_axis_name="c", subcore_axis_name="s")
    partials = pl.kernel(
        _sc_scatter_kernel,
        out_type=jax.ShapeDtypeStruct((3 * 2 * N_NODES,), f32),
        mesh=mesh,
        compiler_params=pltpu.CompilerParams(needs_layout_passes=False,
                                             use_tc_tiling_on_sc=True),
        scratch_types=[
            pltpu.VMEM((16,), f32),
            pltpu.VMEM((16,), f32),
            pltpu.VMEM((16,), f32),
            pltpu.VMEM((CHUNK,), jnp.int32),
            pltpu.VMEM((NSUB, SUB), jnp.int32),
            pltpu.VMEM((CHUNK,), jnp.int32),
            pltpu.VMEM((CHUNK,), jnp.int32),
            pltpu.VMEM((CHUNK,), jnp.int32),
            pltpu.VMEM((CHUNK,), f32),
            pltpu.VMEM((CHUNK,), f32),
            pltpu.VMEM((CHUNK,), f32),
            pltpu.VMEM((CHUNK,), f32),
            pltpu.VMEM((CHUNK,), f32),
            pltpu.VMEM((NSUB, SUB), f32),
            pltpu.VMEM((NSUB, SUB), f32),
            pltpu.VMEM((NSUB, SUB), f32),
            pltpu.VMEM((ROWS_LAST,), f32),
            pltpu.VMEM_SHARED((N_NODES,), f32),
            pltpu.VMEM_SHARED((N_NODES,), f32),
            pltpu.VMEM_SHARED((N_NODES,), f32),
            pltpu.SemaphoreType.DMA,
            pltpu.SemaphoreType.DMA,
        ],
    )(dst1, pair_indices, source_elements, target_elements,
      inverse_distances_sq, ex, ey, ez, a16, b16, cp16)

    p = partials.reshape(3, 2, N_NODES)
    summed = pl.pallas_call(
        _add_body,
        out_shape=jax.ShapeDtypeStruct((3, N_NODES), f32),
    )(p[:, 0, :], p[:, 1, :])
    return summed.T


# speedup vs baseline: 49.8276x; 1.2058x over previous
import jax
import jax.numpy as jnp
from jax import lax
from jax.experimental import pallas as pl
from jax.experimental.pallas import tpu as pltpu
from jax.experimental.pallas import tpu_sc as plsc

N_NODES = 100000
N_EDGES = 3200000

SUB = 80
NSUB = 10
CHUNK = SUB * NSUB
N_CHUNKS = N_EDGES // CHUNK
CHUNKS_PER_W = N_CHUNKS // 32

ROWS_A = 6240
ROWS_LAST = N_NODES - 15 * ROWS_A


def _sc_scatter_kernel(dst1, pair, se, te, inv2, ex, ey, ez, a16, b16, cp16,
                       out,
                       a_v, b_v, cp_v,
                       d1a_v, d1b_v, pra_v, prb_v, sea_v, seb_v, tea_v, teb_v,
                       iva_v, ivb_v, exa_v, exb_v, eya_v, eyb_v, eza_v, ezb_v,
                       dst_v, s_v,
                       updx_v, updy_v, updz_v, zb_v,
                       accx, accy, accz, sem_in0, sem_in1, sem_sc0, sem_sc1):
    f32 = jnp.float32
    cid = lax.axis_index("c")
    sid = lax.axis_index("s")
    wid = cid * 16 + sid
    row0 = sid * ROWS_A
    c0 = wid * CHUNKS_PER_W

    sem_in = (sem_in0, sem_in1)
    sem_sc = (sem_sc0, sem_sc1)
    ins = ((dst1, (d1a_v, d1b_v)), (pair, (pra_v, prb_v)),
           (se, (sea_v, seb_v)), (te, (tea_v, teb_v)),
           (inv2, (iva_v, ivb_v)), (ex, (exa_v, exb_v)),
           (ey, (eya_v, eyb_v)), (ez, (eza_v, ezb_v)))

    def issue_in(i, b):
        e0 = (c0 + i) * CHUNK
        for hbm, bufs in ins:
            pltpu.async_copy(hbm.at[pl.ds(e0, CHUNK)], bufs[b], sem_in[b])

    def wait_in(b):
        for hbm, bufs in ins:
            pltpu.make_async_copy(hbm.at[pl.ds(0, CHUNK)], bufs[b],
                                  sem_in[b]).wait()

    def drain_sc(b):
        for _ in range(3 * NSUB):
            pltpu.make_async_copy(dst1.at[pl.ds(0, SUB)], dst_v.at[0],
                                  sem_sc[b]).wait()

    def do_chunk(b):
        base = b * NSUB
        d1_v = (d1a_v, d1b_v)[b]
        pr_v = (pra_v, prb_v)[b]
        s_e_v = (sea_v, seb_v)[b]
        t_e_v = (tea_v, teb_v)[b]
        iv_v = (iva_v, ivb_v)[b]
        exc_v = (exa_v, exb_v)[b]
        eyc_v = (eya_v, eyb_v)[b]
        ezc_v = (eza_v, ezb_v)[b]
        for j in range(NSUB):
            for g in range(SUB // 16):
                sl = pl.ds(g * 16, 16)
                dst_v[base + j, sl] = d1_v[pl.ds(j * SUB + g * 16, 16)]

        for g in range(CHUNK // 16):
            sl = pl.ds(g * 16, 16)
            pv = pr_v[sl]
            iv = iv_v[sl]
            ci = s_e_v[sl] + s_e_v[sl] + t_e_v[sl]
            av = plsc.load_gather(a_v, [pv])
            bv = plsc.load_gather(b_v, [pv])
            cv = plsc.load_gather(cp_v, [ci])
            s_v[sl] = iv * (bv - cv - av * iv)

        for j in range(NSUB):
            for g in range(SUB // 16):
                sl = pl.ds(g * 16, 16)
                sle = pl.ds(j * SUB + g * 16, 16)
                sv = s_v[sle]
                updx_v[base + j, sl] = sv * exc_v[sle]
                updy_v[base + j, sl] = sv * eyc_v[sle]
                updz_v[base + j, sl] = sv * ezc_v[sle]
            idx = dst_v.at[base + j]
            pltpu.async_copy(updx_v.at[base + j], accx.at[idx],
                             sem_sc[b], add=True)
            pltpu.async_copy(updy_v.at[base + j], accy.at[idx],
                             sem_sc[b], add=True)
            pltpu.async_copy(updz_v.at[base + j], accz.at[idx],
                             sem_sc[b], add=True)

    pltpu.sync_copy(a16, a_v)
    pltpu.sync_copy(b16, b_v)
    pltpu.sync_copy(cp16, cp_v)
    issue_in(0, 0)

    z16 = jnp.zeros((16,), f32)

    def zbody(k, carry):
        zb_v[pl.ds(k * 16, 16)] = z16
        return carry

    lax.fori_loop(0, ROWS_LAST // 16, zbody, 0)

    @pl.when(sid < 15)
    def _():
        for acc in (accx, accy, accz):
            pltpu.sync_copy(zb_v.at[pl.ds(0, ROWS_A)],
                            acc.at[pl.ds(row0, ROWS_A)])

    @pl.when(sid == 15)
    def _():
        for acc in (accx, accy, accz):
            pltpu.sync_copy(zb_v, acc.at[pl.ds(15 * ROWS_A, ROWS_LAST)])

    plsc.subcore_barrier()

    def dbl(k, carry):
        issue_in(2 * k + 1, 1)

        @pl.when(k > 0)
        def _():
            drain_sc(0)

        wait_in(0)
        do_chunk(0)
        issue_in(2 * k + 2, 0)

        @pl.when(k > 0)
        def _():
            drain_sc(1)

        wait_in(1)
        do_chunk(1)
        return carry

    lax.fori_loop(0, (CHUNKS_PER_W - 1) // 2, dbl, 0)
    drain_sc(0)
    wait_in(0)
    do_chunk(0)
    drain_sc(1)
    drain_sc(0)
    plsc.subcore_barrier()

    for m, acc in enumerate((accx, accy, accz)):
        obase = m * (2 * N_NODES) + cid * N_NODES + row0

        @pl.when(sid < 15)
        def _(acc=acc, obase=obase):
            pltpu.sync_copy(acc.at[pl.ds(row0, ROWS_A)],
                            zb_v.at[pl.ds(0, ROWS_A)])
            pltpu.sync_copy(zb_v.at[pl.ds(0, ROWS_A)],
                            out.at[pl.ds(obase, ROWS_A)])

        @pl.when(sid == 15)
        def _(acc=acc, obase=obase):
            pltpu.sync_copy(acc.at[pl.ds(15 * ROWS_A, ROWS_LAST)], zb_v)
            pltpu.sync_copy(zb_v, out.at[pl.ds(obase, ROWS_LAST)])


def _add_body(a_ref, b_ref, o_ref):
    o_ref[...] = a_ref[...] + b_ref[...]


@jax.jit
def kernel(pos, edge_index, pair_indices, source_elements, target_elements,
           inverse_distances_sq, edge_unit_vectors, A, B, raw_charges):
    f32 = jnp.float32
    i32 = jnp.int32
    dst1 = edge_index[0]
    ex = edge_unit_vectors[:, 0]
    ey = edge_unit_vectors[:, 1]
    ez = edge_unit_vectors[:, 2]

    q = raw_charges[0]
    q2 = q * q
    cp16 = jnp.zeros((16,), f32).at[:4].set(jnp.stack([q2, -q2, -q2, q2]))
    a16 = jnp.zeros((16,), f32).at[:3].set(A)
    b16 = jnp.zeros((16,), f32).at[:3].set(B)

    mesh = plsc.VectorSubcoreMesh(core_axis_name="c", subcore_axis_name="s")
    partials = pl.kernel(
        _sc_scatter_kernel,
        out_type=jax.ShapeDtypeStruct((3 * 2 * N_NODES,), f32),
        mesh=mesh,
        compiler_params=pltpu.CompilerParams(needs_layout_passes=False,
                                             use_tc_tiling_on_sc=True),
        scratch_types=[
            pltpu.VMEM((16,), f32),
            pltpu.VMEM((16,), f32),
            pltpu.VMEM((16,), f32),
            pltpu.VMEM((CHUNK,), i32),
            pltpu.VMEM((CHUNK,), i32),
            pltpu.VMEM((CHUNK,), i32),
            pltpu.VMEM((CHUNK,), i32),
            pltpu.VMEM((CHUNK,), i32),
            pltpu.VMEM((CHUNK,), i32),
            pltpu.VMEM((CHUNK,), i32),
            pltpu.VMEM((CHUNK,), i32),
            pltpu.VMEM((CHUNK,), f32),
            pltpu.VMEM((CHUNK,), f32),
            pltpu.VMEM((CHUNK,), f32),
            pltpu.VMEM((CHUNK,), f32),
            pltpu.VMEM((CHUNK,), f32),
            pltpu.VMEM((CHUNK,), f32),
            pltpu.VMEM((CHUNK,), f32),
            pltpu.VMEM((CHUNK,), f32),
            pltpu.VMEM((2 * NSUB, SUB), i32),
            pltpu.VMEM((CHUNK,), f32),
            pltpu.VMEM((2 * NSUB, SUB), f32),
            pltpu.VMEM((2 * NSUB, SUB), f32),
            pltpu.VMEM((2 * NSUB, SUB), f32),
            pltpu.VMEM((ROWS_LAST,), f32),
            pltpu.VMEM_SHARED((N_NODES,), f32),
            pltpu.VMEM_SHARED((N_NODES,), f32),
            pltpu.VMEM_SHARED((N_NODES,), f32),
            pltpu.SemaphoreType.DMA,
            pltpu.SemaphoreType.DMA,
            pltpu.SemaphoreType.DMA,
            pltpu.SemaphoreType.DMA,
        ],
    )(dst1, pair_indices, source_elements, target_elements,
      inverse_distances_sq, ex, ey, ez, a16, b16, cp16)

    p = partials.reshape(3, 2, N_NODES)
    summed = pl.pallas_call(
        _add_body,
        out_shape=jax.ShapeDtypeStruct((3, N_NODES), f32),
    )(p[:, 0, :], p[:, 1, :])
    return summed.T


# speedup vs baseline: 50.1087x; 1.0056x over previous
import jax
import jax.numpy as jnp
from jax import lax
from jax.experimental import pallas as pl
from jax.experimental.pallas import tpu as pltpu
from jax.experimental.pallas import tpu_sc as plsc

N_NODES = 100000
N_EDGES = 3200000

SUB = 80
NSUB = 10
CHUNK = SUB * NSUB
N_CHUNKS = N_EDGES // CHUNK
CHUNKS_PER_W = N_CHUNKS // 32

ROWS_A = 6240
ROWS_LAST = N_NODES - 15 * ROWS_A


def _sc_scatter_kernel(dst1, pair, se, te, inv2, ex, ey, ez, a16, b16, cp16,
                       out,
                       a_v, b_v, cp_v,
                       d1a_v, d1b_v, pra_v, prb_v, sea_v, seb_v, tea_v, teb_v,
                       iva_v, ivb_v, exa_v, exb_v, eya_v, eyb_v, eza_v, ezb_v,
                       dst_v, s_v,
                       updx_v, updy_v, updz_v, zb_v,
                       accx, accy, accz, sem_in0, sem_in1, sem_sc0, sem_sc1):
    f32 = jnp.float32
    cid = lax.axis_index("c")
    sid = lax.axis_index("s")
    wid = cid * 16 + sid
    row0 = sid * ROWS_A
    c0 = wid * CHUNKS_PER_W

    sem_in = (sem_in0, sem_in1)
    sem_sc = (sem_sc0, sem_sc1)
    ins = ((dst1, (d1a_v, d1b_v)), (pair, (pra_v, prb_v)),
           (se, (sea_v, seb_v)), (te, (tea_v, teb_v)),
           (inv2, (iva_v, ivb_v)), (ex, (exa_v, exb_v)),
           (ey, (eya_v, eyb_v)), (ez, (eza_v, ezb_v)))

    def issue_in(i, b):
        e0 = (c0 + i) * CHUNK
        for hbm, bufs in ins:
            pltpu.async_copy(hbm.at[pl.ds(e0, CHUNK)], bufs[b], sem_in[b])

    def wait_in(b):
        for hbm, bufs in ins:
            pltpu.make_async_copy(hbm.at[pl.ds(0, CHUNK)], bufs[b],
                                  sem_in[b]).wait()

    def drain_sc(b):
        for _ in range(3 * NSUB):
            pltpu.make_async_copy(dst1.at[pl.ds(0, SUB)], dst_v.at[0],
                                  sem_sc[b]).wait()

    def do_chunk(b):
        base = b * NSUB
        d1_v = (d1a_v, d1b_v)[b]
        pr_v = (pra_v, prb_v)[b]
        s_e_v = (sea_v, seb_v)[b]
        t_e_v = (tea_v, teb_v)[b]
        iv_v = (iva_v, ivb_v)[b]
        exc_v = (exa_v, exb_v)[b]
        eyc_v = (eya_v, eyb_v)[b]
        ezc_v = (eza_v, ezb_v)[b]
        for g in range(CHUNK // 16):
            sl = pl.ds(g * 16, 16)
            pv = pr_v[sl]
            iv = iv_v[sl]
            ci = s_e_v[sl] + s_e_v[sl] + t_e_v[sl]
            av = plsc.load_gather(a_v, [pv])
            bv = plsc.load_gather(b_v, [pv])
            cv = plsc.load_gather(cp_v, [ci])
            s_v[sl] = iv * (bv - cv - av * iv)

        for j in range(NSUB):
            for g in range(SUB // 16):
                sl = pl.ds(g * 16, 16)
                sle = pl.ds(j * SUB + g * 16, 16)
                sv = s_v[sle]
                dst_v[base + j, sl] = d1_v[sle]
                updx_v[base + j, sl] = sv * exc_v[sle]
                updy_v[base + j, sl] = sv * eyc_v[sle]
                updz_v[base + j, sl] = sv * ezc_v[sle]
            idx = dst_v.at[base + j]
            pltpu.async_copy(updx_v.at[base + j], accx.at[idx],
                             sem_sc[b], add=True)
            pltpu.async_copy(updy_v.at[base + j], accy.at[idx],
                             sem_sc[b], add=True)
            pltpu.async_copy(updz_v.at[base + j], accz.at[idx],
                             sem_sc[b], add=True)

    pltpu.sync_copy(a16, a_v)
    pltpu.sync_copy(b16, b_v)
    pltpu.sync_copy(cp16, cp_v)
    issue_in(0, 0)

    z16 = jnp.zeros((16,), f32)

    def zbody(k, carry):
        zb_v[pl.ds(k * 16, 16)] = z16
        return carry

    lax.fori_loop(0, ROWS_LAST // 16, zbody, 0)

    @pl.when(sid < 15)
    def _():
        for acc in (accx, accy, accz):
            pltpu.sync_copy(zb_v.at[pl.ds(0, ROWS_A)],
                            acc.at[pl.ds(row0, ROWS_A)])

    @pl.when(sid == 15)
    def _():
        for acc in (accx, accy, accz):
            pltpu.sync_copy(zb_v, acc.at[pl.ds(15 * ROWS_A, ROWS_LAST)])

    plsc.subcore_barrier()

    def dbl(k, carry):
        issue_in(2 * k + 1, 1)

        @pl.when(k > 0)
        def _():
            drain_sc(0)

        wait_in(0)
        do_chunk(0)
        issue_in(2 * k + 2, 0)

        @pl.when(k > 0)
        def _():
            drain_sc(1)

        wait_in(1)
        do_chunk(1)
        return carry

    lax.fori_loop(0, (CHUNKS_PER_W - 1) // 2, dbl, 0)
    drain_sc(0)
    wait_in(0)
    do_chunk(0)
    drain_sc(1)
    drain_sc(0)
    plsc.subcore_barrier()

    for m, acc in enumerate((accx, accy, accz)):
        obase = m * (2 * N_NODES) + cid * N_NODES + row0

        @pl.when(sid < 15)
        def _(acc=acc, obase=obase):
            pltpu.sync_copy(acc.at[pl.ds(row0, ROWS_A)],
                            zb_v.at[pl.ds(0, ROWS_A)])
            pltpu.sync_copy(zb_v.at[pl.ds(0, ROWS_A)],
                            out.at[pl.ds(obase, ROWS_A)])

        @pl.when(sid == 15)
        def _(acc=acc, obase=obase):
            pltpu.sync_copy(acc.at[pl.ds(15 * ROWS_A, ROWS_LAST)], zb_v)
            pltpu.sync_copy(zb_v, out.at[pl.ds(obase, ROWS_LAST)])


def _add_body(a_ref, b_ref, o_ref):
    o_ref[...] = a_ref[...] + b_ref[...]


@jax.jit
def kernel(pos, edge_index, pair_indices, source_elements, target_elements,
           inverse_distances_sq, edge_unit_vectors, A, B, raw_charges):
    f32 = jnp.float32
    i32 = jnp.int32
    dst1 = edge_index[0]
    ex = edge_unit_vectors[:, 0]
    ey = edge_unit_vectors[:, 1]
    ez = edge_unit_vectors[:, 2]

    q = raw_charges[0]
    q2 = q * q
    cp16 = jnp.zeros((16,), f32).at[:4].set(jnp.stack([q2, -q2, -q2, q2]))
    a16 = jnp.zeros((16,), f32).at[:3].set(A)
    b16 = jnp.zeros((16,), f32).at[:3].set(B)

    mesh = plsc.VectorSubcoreMesh(core_axis_name="c", subcore_axis_name="s")
    partials = pl.kernel(
        _sc_scatter_kernel,
        out_type=jax.ShapeDtypeStruct((3 * 2 * N_NODES,), f32),
        mesh=mesh,
        compiler_params=pltpu.CompilerParams(needs_layout_passes=False,
                                             use_tc_tiling_on_sc=True),
        scratch_types=[
            pltpu.VMEM((16,), f32),
            pltpu.VMEM((16,), f32),
            pltpu.VMEM((16,), f32),
            pltpu.VMEM((CHUNK,), i32),
            pltpu.VMEM((CHUNK,), i32),
            pltpu.VMEM((CHUNK,), i32),
            pltpu.VMEM((CHUNK,), i32),
            pltpu.VMEM((CHUNK,), i32),
            pltpu.VMEM((CHUNK,), i32),
            pltpu.VMEM((CHUNK,), i32),
            pltpu.VMEM((CHUNK,), i32),
            pltpu.VMEM((CHUNK,), f32),
            pltpu.VMEM((CHUNK,), f32),
            pltpu.VMEM((CHUNK,), f32),
            pltpu.VMEM((CHUNK,), f32),
            pltpu.VMEM((CHUNK,), f32),
            pltpu.VMEM((CHUNK,), f32),
            pltpu.VMEM((CHUNK,), f32),
            pltpu.VMEM((CHUNK,), f32),
            pltpu.VMEM((2 * NSUB, SUB), i32),
            pltpu.VMEM((CHUNK,), f32),
            pltpu.VMEM((2 * NSUB, SUB), f32),
            pltpu.VMEM((2 * NSUB, SUB), f32),
            pltpu.VMEM((2 * NSUB, SUB), f32),
            pltpu.VMEM((ROWS_LAST,), f32),
            pltpu.VMEM_SHARED((N_NODES,), f32),
            pltpu.VMEM_SHARED((N_NODES,), f32),
            pltpu.VMEM_SHARED((N_NODES,), f32),
            pltpu.SemaphoreType.DMA,
            pltpu.SemaphoreType.DMA,
            pltpu.SemaphoreType.DMA,
            pltpu.SemaphoreType.DMA,
        ],
    )(dst1, pair_indices, source_elements, target_elements,
      inverse_distances_sq, ex, ey, ez, a16, b16, cp16)

    p = partials.reshape(3, 2, N_NODES)
    summed = pl.pallas_call(
        _add_body,
        out_shape=jax.ShapeDtypeStruct((3, N_NODES), f32),
    )(p[:, 0, :], p[:, 1, :])
    return summed.T
